# baseline (speedup 1.0000x reference)
import functools

import jax
import jax.numpy as jnp
from jax import lax
from jax.experimental import pallas as pl
from jax.experimental.pallas import tpu as pltpu
from jax.experimental.pallas import tpu_sc as plsc

D = 64
NW = 32
G = 128
L = 16


def _make_sc_emb(hist: int, batch: int):
    nb = batch // NW
    mesh = plsc.VectorSubcoreMesh(core_axis_name="c", subcore_axis_name="s")

    @functools.partial(
        pl.kernel,
        out_type=jax.ShapeDtypeStruct((hist, D, batch), jnp.float32),
        mesh=mesh,
        compiler_params=pltpu.CompilerParams(
            use_tc_tiling_on_sc=True, needs_layout_passes=False),
        scratch_types=[
            pltpu.VMEM((G,), jnp.int32),
            pltpu.VMEM((G,), jnp.int32),
            pltpu.VMEM((G,), jnp.int32),
            pltpu.VMEM((G,), jnp.int32),
            pltpu.VMEM((G,), jnp.int32),
            pltpu.VMEM((G, 2 * D), jnp.float32),
            pltpu.VMEM((G, 2 * D), jnp.float32),
            pltpu.VMEM((D, G), jnp.float32),
            pltpu.SemaphoreType.DMA,
            pltpu.SemaphoreType.DMA,
        ],
    )
    def emb(tok_hbm, w2_hbm, out_hbm, tokbuf, widx0, widx1, par0, par1,
            wide0, wide1, outblk, sem0, sem1):
        wid = lax.axis_index("s") * 2 + lax.axis_index("c")
        b0 = wid * nb
        iota = lax.iota(jnp.int32, L)

        def load_idx(h, widx, par):
            pltpu.sync_copy(tok_hbm.at[h, pl.ds(b0, G)], tokbuf)
            for i in range(G // L):
                t = tokbuf[pl.ds(i * L, L)]
                widx[pl.ds(i * L, L)] = lax.shift_right_logical(t, 1)
                par[pl.ds(i * L, L)] = lax.shift_left(
                    lax.bitwise_and(t, 1), 6)

        def transpose_select(h, wide, par):
            @pl.loop(0, D)
            def _(d):
                for g in range(G // L):
                    jv = iota + (g * L)
                    pv = par[pl.ds(g * L, L)]
                    val = plsc.load_gather(wide, [jv, pv + d])
                    plsc.store_scatter(
                        outblk, [jnp.broadcast_to(d, (L,)), jv], val)
            pltpu.sync_copy(outblk, out_hbm.at[h, :, pl.ds(b0, G)])

        load_idx(0, widx0, par0)
        pltpu.async_copy(w2_hbm.at[widx0], wide0, sem0)

        @pl.loop(0, hist, step=2)
        def _(h):
            load_idx(h + 1, widx1, par1)
            pltpu.make_async_copy(w2_hbm.at[widx0], wide0, sem0).wait()
            pltpu.async_copy(w2_hbm.at[widx1], wide1, sem1)
            transpose_select(h, wide0, par0)
            pltpu.make_async_copy(w2_hbm.at[widx1], wide1, sem1).wait()

            @pl.when(h + 2 < hist)
            def _():
                load_idx(h + 2, widx0, par0)
                pltpu.async_copy(w2_hbm.at[widx0], wide0, sem0)

            transpose_select(h + 1, wide1, par1)

    return emb


def kernel(tokens, weights):
    batch, hist = tokens.shape
    vocab = weights.shape[0]
    tok_t = tokens.T.astype(jnp.int32)
    w2 = weights.reshape(vocab // 2, 2 * D)
    out_t = _make_sc_emb(hist, batch)(tok_t, w2)
    return out_t.transpose(2, 0, 1)


# speedup vs baseline: 1.5204x; 1.5204x over previous
import functools

import jax
import jax.numpy as jnp
from jax import lax
from jax.experimental import pallas as pl
from jax.experimental.pallas import tpu as pltpu
from jax.experimental.pallas import tpu_sc as plsc

D = 64
NW = 32
G = 128


def _make_sc_emb(hist: int, batch: int):
    mesh = plsc.VectorSubcoreMesh(core_axis_name="c", subcore_axis_name="s")

    @functools.partial(
        pl.kernel,
        out_type=jax.ShapeDtypeStruct((hist, batch, 2 * D), jnp.float32),
        mesh=mesh,
        compiler_params=pltpu.CompilerParams(
            use_tc_tiling_on_sc=True, needs_layout_passes=False),
        scratch_types=[
            pltpu.VMEM((hist, G), jnp.int32),
            pltpu.VMEM((G, 2 * D), jnp.float32),
            pltpu.VMEM((G, 2 * D), jnp.float32),
            pltpu.SemaphoreType.DMA,
            pltpu.SemaphoreType.DMA,
        ],
    )
    def emb(tok_hbm, w_hbm, out_hbm, tokbuf, wide0, wide1, sem0, sem1):
        wid = lax.axis_index("s") * 2 + lax.axis_index("c")
        b0 = wid * G
        pltpu.sync_copy(tok_hbm.at[:, pl.ds(b0, G)], tokbuf)
        pltpu.async_copy(w_hbm.at[tokbuf.at[0]], wide0, sem0)

        @pl.loop(0, hist, step=2)
        def _(h):
            pltpu.make_async_copy(w_hbm.at[tokbuf.at[h]], wide0, sem0).wait()
            pltpu.async_copy(w_hbm.at[tokbuf.at[h + 1]], wide1, sem1)
            pltpu.sync_copy(wide0, out_hbm.at[h, pl.ds(b0, G), :])
            pltpu.make_async_copy(
                w_hbm.at[tokbuf.at[h + 1]], wide1, sem1).wait()

            @pl.when(h + 2 < hist)
            def _():
                pltpu.async_copy(w_hbm.at[tokbuf.at[h + 2]], wide0, sem0)

            pltpu.sync_copy(wide1, out_hbm.at[h + 1, pl.ds(b0, G), :])

    return emb


def kernel(tokens, weights):
    batch, hist = tokens.shape
    tok_t = tokens.T.astype(jnp.int32)
    w_pad = jnp.pad(weights, ((0, 0), (0, D)))
    out = _make_sc_emb(hist, batch)(tok_t, w_pad)
    return out[..., :D].transpose(1, 0, 2)
